# baseline (speedup 1.0000x reference)
import functools

import jax
import jax.numpy as jnp
from jax import lax
from jax.experimental import pallas as pl
from jax.experimental.pallas import tpu as pltpu
from jax.experimental.pallas import tpu_sc as plsc

DIM = 128
NBLK = DIM // 16
GMAX = 128
STEP = 256
GPS = STEP // 16
NBUF = 2
NBUF2 = 4


def _make_kernel(batch):
    info = plsc.get_sparse_core_info()
    nc, ns = info.num_cores, info.num_subcores
    nw = nc * ns
    bags_per_w = batch // nw
    off_len = bags_per_w + 16

    mesh = plsc.VectorSubcoreMesh(core_axis_name="c", subcore_axis_name="s")

    @functools.partial(
        pl.kernel,
        out_type=jax.ShapeDtypeStruct((batch, DIM), jnp.float32),
        mesh=mesh,
        scratch_types=[
            pltpu.VMEM((off_len,), jnp.int32),
            pltpu.VMEM((NBUF2 * STEP,), jnp.int32),
            pltpu.VMEM((NBUF2 * STEP,), jnp.float32),
            pltpu.VMEM((NBUF * STEP, DIM), jnp.float32),
            pltpu.VMEM((bags_per_w, DIM), jnp.float32),
            pltpu.SemaphoreType.DMA((NBUF2,)),
            pltpu.SemaphoreType.DMA((NBUF,)),
        ],
    )
    def kern(idx_hbm, w_hbm, off_hbm, tab_hbm, out_hbm,
             off_v, idx_v, w_v, rows_v, out_v, iwsem, gsem):
        wid = lax.axis_index("s") * nc + lax.axis_index("c")
        bag0 = wid * bags_per_w
        pltpu.sync_copy(off_hbm.at[pl.ds(bag0, off_len)], off_v)

        head = off_v[pl.ds(0, 16)]
        start_w = head[0]
        e0 = head[1]
        wbase = pl.multiple_of(start_w - lax.rem(start_w, 8), 8)
        tail_v = off_v[pl.ds(bags_per_w, 16)]
        end_w = tail_v[0]
        nch = (end_w - wbase + (STEP - 1)) // STEP

        def issue_iw(j):
            slot = lax.rem(j, NBUF2)
            pos = pl.multiple_of(wbase + j * STEP, 8)
            dst = pl.ds(slot * STEP, STEP)
            pltpu.async_copy(idx_hbm.at[pl.ds(pos, STEP)], idx_v.at[dst],
                             iwsem.at[slot])
            pltpu.async_copy(w_hbm.at[pl.ds(pos, STEP)], w_v.at[dst],
                             iwsem.at[slot])

        def wait_iw(j):
            slot = lax.rem(j, NBUF2)
            dst = pl.ds(slot * STEP, STEP)
            pltpu.make_async_copy(idx_hbm.at[pl.ds(0, STEP)], idx_v.at[dst],
                                  iwsem.at[slot]).wait()
            pltpu.make_async_copy(w_hbm.at[pl.ds(0, STEP)], w_v.at[dst],
                                  iwsem.at[slot]).wait()

        def issue_gather(j):
            slot = lax.rem(j, NBUF)
            slot2 = lax.rem(j, NBUF2)
            for h in range(STEP // GMAX):
                pltpu.async_copy(
                    tab_hbm.at[idx_v.at[pl.ds(slot2 * STEP + h * GMAX, GMAX)]],
                    rows_v.at[pl.ds(slot * STEP + h * GMAX, GMAX), :],
                    gsem.at[slot])

        def wait_gather(j):
            slot = lax.rem(j, NBUF)
            pltpu.make_async_copy(tab_hbm.at[pl.ds(0, STEP), :],
                                  rows_v.at[pl.ds(slot * STEP, STEP), :],
                                  gsem.at[slot]).wait()

        def need(c, iw, gi, gw):
            iw_hi = jnp.minimum(c + 2 * NBUF, nch)
            lax.fori_loop(iw, iw_hi, lambda j, _: (issue_iw(j), 0)[1], 0)
            iw = jnp.maximum(iw, iw_hi)

            gi_hi = jnp.minimum(c + NBUF, nch)
            lax.fori_loop(
                gi, gi_hi,
                lambda j, _: (wait_iw(j), issue_gather(j), 0)[2], 0)
            gi = jnp.maximum(gi, gi_hi)

            lax.fori_loop(gw, c + 1, lambda j, _: (wait_gather(j), 0)[1], 0)
            gw = jnp.maximum(gw, c + 1)
            return iw, gi, gw

        def need_if(c, iw, gi, gw):
            return lax.cond(gw <= c, lambda: need(c, iw, gi, gw),
                            lambda: (iw, gi, gw))

        lanes = lax.iota(jnp.int32, 16)

        def group_addrs(g):
            c = g // GPS
            go = g - c * GPS
            woff = lax.rem(c, NBUF2) * STEP + go * 16
            rbase = lax.rem(c, NBUF) * STEP + go * 16
            return c, woff, rbase

        def dual_group(wm_a, wm_b, rbase, acc, car):
            acc = list(acc)
            car = list(car)
            for e16 in range(16):
                wa = jnp.zeros((16,), jnp.float32) + wm_a[e16]
                wb = jnp.zeros((16,), jnp.float32) + wm_b[e16]
                row = rbase + e16
                for d in range(NBLK):
                    blk = rows_v[row, pl.ds(d * 16, 16)]
                    acc[d] = acc[d] + wa * blk
                    car[d] = car[d] + wb * blk
            return tuple(acc), tuple(car)

        zacc = tuple(jnp.zeros((16,), jnp.float32) for _ in range(NBLK))

        zrow = jnp.zeros((16,), jnp.float32)
        for r in range(16):
            for d in range(NBLK):
                rows_v[r, pl.ds(d * 16, 16)] = zrow

        z = jnp.int32(0)
        pro = need_if(jnp.where(nch > 0, 0, -1), z, z, z)
        gp = (start_w - wbase) // 16
        glo0 = (start_w - wbase + 15) // 16
        _, woff0, rbase0 = group_addrs(gp)
        wvec0 = w_v[pl.ds(pl.multiple_of(woff0, 16), 16)]
        p0 = wbase + gp * 16 + lanes
        lim0 = jnp.minimum(e0, wbase + 16 * glo0)
        wm0 = jnp.where((p0 >= start_w) & (p0 < lim0), wvec0, 0.0)
        _, car0 = dual_group(wm0, wm0, rbase0, zacc, zacc)

        def bag_body(b, st):
            car = st[:NBLK]
            iw, gi, gw = st[NBLK:]
            offv = off_v[pl.ds(b, 16)]
            s = offv[0]
            e = offv[1]
            e2 = offv[2]
            g_lo = (s - wbase + 15) // 16
            ge1 = (e - wbase + 15) // 16
            gt = ge1 - 1

            def g_body(g, carry):
                acc = list(carry[:NBLK])
                iw, gi, gw = carry[NBLK:]
                c, woff, rbase = group_addrs(g)
                iw, gi, gw = need_if(c, iw, gi, gw)
                wvec = w_v[pl.ds(pl.multiple_of(woff, 16), 16)]
                for e16 in range(16):
                    w_b = jnp.zeros((16,), jnp.float32) + wvec[e16]
                    row = rbase + e16
                    for d in range(NBLK):
                        acc[d] = acc[d] + w_b * rows_v[row, pl.ds(d * 16, 16)]
                return tuple(acc) + (iw, gi, gw)

            res = lax.fori_loop(g_lo, jnp.maximum(gt, g_lo), g_body,
                                car + (iw, gi, gw))
            acc = res[:NBLK]
            iw, gi, gw = res[NBLK:]
            iw, gi, gw = need_if(gt // GPS, iw, gi, gw)

            gt_c = jnp.maximum(gt, z)
            c, woff, rbase = group_addrs(gt_c)
            wvec = w_v[pl.ds(pl.multiple_of(woff, 16), 16)]
            p = wbase + gt_c * 16 + lanes
            e_eff = jnp.where(gt >= g_lo, e, jnp.int32(-1))
            e2_eff = jnp.where(gt >= 0, e2, jnp.int32(-1))
            wm_s = jnp.where(p < e_eff, wvec, 0.0)
            wm_n = jnp.where((p >= e) & (p < e2_eff), wvec, 0.0)
            nacc, ncar = dual_group(wm_s, wm_n, rbase, acc, zacc)
            for d in range(NBLK):
                out_v[b, pl.ds(d * 16, 16)] = nacc[d]
            return ncar + (iw, gi, gw)

        lax.fori_loop(0, bags_per_w, bag_body, car0 + pro)
        pltpu.sync_copy(out_v, out_hbm.at[pl.ds(bag0, bags_per_w), :])

    return kern


@jax.jit
def kernel(indices, weights, offsets, table):
    batch = offsets.shape[0] - 1
    total = indices.shape[0]
    pad = 2 * STEP
    idx_p = jnp.pad(indices.astype(jnp.int32), (0, pad))
    w_p = jnp.pad(weights, (0, pad))
    off_p = jnp.pad(offsets, (0, 256), constant_values=total)
    kern = _make_kernel(batch)
    return kern(idx_p, w_p, off_p, table)
